# baseline (speedup 1.0000x reference)
import functools

import jax
import jax.numpy as jnp
from jax import lax
from jax.experimental import pallas as pl
from jax.experimental.pallas import tpu as pltpu
from jax.experimental.pallas import tpu_sc as plsc

_B = 16384
_L = 50
_N = _B * _L
_PACK = 4
_LINES = _N // _PACK
_CHUNK_LINES = 128
_NCHUNKS = _LINES // _CHUNK_LINES
_NW = 32
_CHUNKS_PER_W = _NCHUNKS // _NW


def _sc_gather_packed(emb1, emb2, x4, y4):
    mesh = plsc.VectorSubcoreMesh(core_axis_name="c", subcore_axis_name="s")

    @functools.partial(
        pl.kernel,
        out_type=(
            jax.ShapeDtypeStruct((_LINES, 128), jnp.float32),
            jax.ShapeDtypeStruct((_LINES, 128), jnp.float32),
        ),
        mesh=mesh,
        scratch_types=[
            pltpu.VMEM((_PACK, _CHUNK_LINES), jnp.int32),
            pltpu.VMEM((_PACK, _CHUNK_LINES, 32), jnp.float32),
            pltpu.SemaphoreType.DMA,
        ],
        compiler_params=pltpu.CompilerParams(use_tc_tiling_on_sc=False),
    )
    def gather_kernel(t1_ref, t2_ref, x4_ref, y4_ref, o1_ref, o2_ref,
                      idx_v, rows_v, sem):
        wid = lax.axis_index("s") * 2 + lax.axis_index("c")
        chunk0 = wid * _CHUNKS_PER_W

        def make_body(tab_ref, ig_ref, out_ref):
            def body(i, carry):
                line0 = (chunk0 + i) * _CHUNK_LINES
                pltpu.sync_copy(
                    ig_ref.at[:, pl.ds(line0, _CHUNK_LINES)], idx_v)
                descs = [
                    pltpu.async_copy(
                        tab_ref.at[idx_v.at[j]], rows_v.at[j], sem)
                    for j in range(_PACK)
                ]
                for d in descs:
                    d.wait()
                for j in range(_PACK):
                    pltpu.sync_copy(
                        rows_v.at[j],
                        out_ref.at[pl.ds(line0, _CHUNK_LINES),
                                   pl.ds(32 * j, 32)])
                return carry
            return body

        lax.fori_loop(0, _CHUNKS_PER_W, make_body(t1_ref, x4_ref, o1_ref), 0)
        lax.fori_loop(0, _CHUNKS_PER_W, make_body(t2_ref, y4_ref, o2_ref), 0)

    return gather_kernel(emb1, emb2, x4, y4)


def _tc_linear_relu_packed(gw, i4, w_t):
    out_dim = w_t.shape[1]
    lines_per_blk = 1024
    grid = _LINES // lines_per_blk

    bd = jnp.zeros((_PACK, 128, out_dim), jnp.float32)
    for j in range(_PACK):
        bd = bd.at[j, 32 * j:32 * (j + 1), :].set(w_t)

    def mm_kernel(g_ref, i_ref, w_ref, o_ref):
        gwb = g_ref[...]
        mt = jnp.transpose(i_ref[...], (1, 0)) != 0
        for j in range(_PACK):
            o = jnp.maximum(
                jnp.dot(gwb, w_ref[j], preferred_element_type=jnp.float32),
                0.0)
            o_ref[:, pl.ds(out_dim * j, out_dim)] = jnp.where(
                mt[:, j:j + 1], o, 0.0)

    return pl.pallas_call(
        mm_kernel,
        grid=(grid,),
        in_specs=[
            pl.BlockSpec((lines_per_blk, 128), lambda i: (i, 0)),
            pl.BlockSpec((_PACK, lines_per_blk), lambda i: (0, i)),
            pl.BlockSpec((_PACK, 128, out_dim), lambda i: (0, 0, 0)),
        ],
        out_specs=pl.BlockSpec((lines_per_blk, _PACK * out_dim),
                               lambda i: (i, 0)),
        out_shape=jax.ShapeDtypeStruct((_LINES, _PACK * out_dim),
                                       jnp.float32),
    )(gw, i4, bd)


def kernel(x, y, emb1, emb2, W1, W2):
    out_dim = W1.shape[0]
    x4 = x.reshape(_PACK, _LINES)
    y4 = y.reshape(_PACK, _LINES)

    g1, g2 = _sc_gather_packed(emb1, emb2, x4, y4)

    p1 = _tc_linear_relu_packed(g1, x4, W1.T)
    p2 = _tc_linear_relu_packed(g2, y4, W2.T)

    o1 = jnp.transpose(p1.reshape(_LINES, _PACK, out_dim),
                       (1, 0, 2)).reshape(_B, _L, out_dim)
    o2 = jnp.transpose(p2.reshape(_LINES, _PACK, out_dim),
                       (1, 0, 2)).reshape(_B, _L, out_dim)
    return (o1, o2)


# speedup vs baseline: 1.7988x; 1.7988x over previous
import functools

import jax
import jax.numpy as jnp
from jax import lax
from jax.experimental import pallas as pl
from jax.experimental.pallas import tpu as pltpu
from jax.experimental.pallas import tpu_sc as plsc

_B = 16384
_L = 50
_N = _B * _L
_V2 = 100000
_D = 32
_O = 64
_NW = 32

_P1 = 4
_LINES1 = _N // _P1
_CH1 = _LINES1 // 128
_CH1_W = _CH1 // _NW

_P2 = 2
_LINES2 = _N // _P2
_CH2 = _LINES2 // 128
_CH2_W = _CH2 // _NW


def _deinterleave(src_ref, dst_ref, stride, nstreams):
    for j in range(nstreams):
        for g in range(8):
            il = lax.iota(jnp.int32, 16) + 16 * g
            f = stride * il + j
            vals = plsc.load_gather(
                src_ref, [lax.shift_right_logical(f, 7),
                          lax.bitwise_and(f, 127)])
            dst_ref[j, pl.ds(16 * g, 16)] = vals


def _sc_gather(emb1, p2, x2, y2):
    mesh = plsc.VectorSubcoreMesh(core_axis_name="c", subcore_axis_name="s")

    @functools.partial(
        pl.kernel,
        out_type=(
            jax.ShapeDtypeStruct((_LINES1, 128), jnp.float32),
            jax.ShapeDtypeStruct((_LINES2, 128), jnp.float32),
        ),
        mesh=mesh,
        scratch_types=[
            pltpu.VMEM((_P1, 128), jnp.int32),
            pltpu.VMEM((_P1, 128), jnp.int32),
            pltpu.VMEM((_P2, 128), jnp.int32),
            pltpu.VMEM((_P2, 128), jnp.int32),
            pltpu.VMEM((_P1, 128, _D), jnp.float32),
            pltpu.VMEM((_P2, 128, _O), jnp.float32),
            pltpu.SemaphoreType.DMA,
        ],
        compiler_params=pltpu.CompilerParams(use_tc_tiling_on_sc=False,
                                             needs_layout_passes=False),
    )
    def gather_kernel(t1_ref, p2_ref, x2_ref, y2_ref, o1_ref, o2_ref,
                      nat_v, idx_v, nat2_v, idx2_v, rows1_v, rows2_v, sem):
        wid = lax.axis_index("s") * 2 + lax.axis_index("c")

        def body1(i, carry):
            c = wid * _CH1_W + i
            line0 = c * 128
            pltpu.sync_copy(x2_ref.at[pl.ds(_P1 * c, _P1)], nat_v)
            _deinterleave(nat_v, idx_v, _P1, _P1)
            descs = [
                pltpu.async_copy(t1_ref.at[idx_v.at[j]], rows1_v.at[j], sem)
                for j in range(_P1)
            ]
            for d in descs:
                d.wait()
            for j in range(_P1):
                pltpu.sync_copy(
                    rows1_v.at[j],
                    o1_ref.at[pl.ds(line0, 128), pl.ds(_D * j, _D)])
            return carry

        def body2(i, carry):
            c = wid * _CH2_W + i
            line0 = c * 128
            pltpu.sync_copy(y2_ref.at[pl.ds(_P2 * c, _P2)], nat2_v)
            _deinterleave(nat2_v, idx2_v, _P2, _P2)
            descs = [
                pltpu.async_copy(p2_ref.at[idx2_v.at[j]], rows2_v.at[j], sem)
                for j in range(_P2)
            ]
            for d in descs:
                d.wait()
            for j in range(_P2):
                pltpu.sync_copy(
                    rows2_v.at[j],
                    o2_ref.at[pl.ds(line0, 128), pl.ds(_O * j, _O)])
            return carry

        lax.fori_loop(0, _CH1_W, body1, 0)
        lax.fori_loop(0, _CH2_W, body2, 0)

    return gather_kernel(emb1, p2, x2, y2)


def _tc_precompute_p2(emb2, w_t):
    blk = 10000
    grid = _V2 // blk

    def p2_kernel(e_ref, w_ref, o_ref):
        o = jnp.maximum(
            jnp.dot(e_ref[...], w_ref[...],
                    preferred_element_type=jnp.float32), 0.0)
        rid = lax.broadcasted_iota(jnp.int32, (blk, _O), 0)
        first = pl.program_id(0) == 0
        o_ref[...] = jnp.where(jnp.logical_and(first, rid == 0), 0.0, o)

    return pl.pallas_call(
        p2_kernel,
        grid=(grid,),
        in_specs=[
            pl.BlockSpec((blk, _D), lambda i: (i, 0)),
            pl.BlockSpec((_D, _O), lambda i: (0, 0)),
        ],
        out_specs=pl.BlockSpec((blk, _O), lambda i: (i, 0)),
        out_shape=jax.ShapeDtypeStruct((_V2, _O), jnp.float32),
    )(emb2, w_t)


def _tc_linear_relu_packed(gw, w_t):
    lines_per_blk = 1024
    grid = _LINES1 // lines_per_blk

    bd = jnp.zeros((_P1, 128, _O), jnp.float32)
    for j in range(_P1):
        bd = bd.at[j, _D * j:_D * (j + 1), :].set(w_t)

    def mm_kernel(g_ref, w_ref, o_ref):
        gwb = g_ref[...]
        for j in range(_P1):
            o_ref[:, pl.ds(_O * j, _O)] = jnp.maximum(
                jnp.dot(gwb, w_ref[j], preferred_element_type=jnp.float32),
                0.0)

    return pl.pallas_call(
        mm_kernel,
        grid=(grid,),
        in_specs=[
            pl.BlockSpec((lines_per_blk, 128), lambda i: (i, 0)),
            pl.BlockSpec((_P1, 128, _O), lambda i: (0, 0, 0)),
        ],
        out_specs=pl.BlockSpec((lines_per_blk, _P1 * _O), lambda i: (i, 0)),
        out_shape=jax.ShapeDtypeStruct((_LINES1, _P1 * _O), jnp.float32),
    )(gw, bd)


def kernel(x, y, emb1, emb2, W1, W2):
    x2 = x.reshape(_N // 128, 128)
    y2 = y.reshape(_N // 128, 128)

    p2_table = _tc_precompute_p2(emb2, W2.T)
    g1, q2 = _sc_gather(emb1, p2_table, x2, y2)

    p1 = _tc_linear_relu_packed(g1, W1.T)

    o1 = jnp.where((x != 0)[:, :, None],
                   p1.reshape(_B, _L, _O), 0.0)
    o2 = q2.reshape(_B, _L, _O)
    return (o1, o2)


# speedup vs baseline: 1.9920x; 1.1074x over previous
import functools

import jax
import jax.numpy as jnp
from jax import lax
from jax.experimental import pallas as pl
from jax.experimental.pallas import tpu as pltpu
from jax.experimental.pallas import tpu_sc as plsc

_B = 16384
_L = 50
_N = _B * _L
_V2 = 100000
_D = 32
_O = 64
_NW = 32

_P1 = 4
_LINES1 = _N // _P1
_CH1 = _LINES1 // 128
_CH1_W = _CH1 // _NW

_P2 = 2
_LINES2 = _N // _P2
_CH2 = _LINES2 // 128
_CH2_W = _CH2 // _NW


def _deinterleave(src_ref, dst_ref, stride, nstreams):
    for j in range(nstreams):
        for g in range(8):
            il = lax.iota(jnp.int32, 16) + 16 * g
            f = stride * il + j
            vals = plsc.load_gather(
                src_ref, [lax.shift_right_logical(f, 7),
                          lax.bitwise_and(f, 127)])
            dst_ref[j, pl.ds(16 * g, 16)] = vals


def _sc_gather(emb1, p2, x2, y2):
    mesh = plsc.VectorSubcoreMesh(core_axis_name="c", subcore_axis_name="s")

    @functools.partial(
        pl.kernel,
        out_type=(
            jax.ShapeDtypeStruct((_LINES1, 128), jnp.float32),
            jax.ShapeDtypeStruct((_LINES2, 128), jnp.float32),
            jax.ShapeDtypeStruct((_P1, _LINES1), jnp.int32),
        ),
        mesh=mesh,
        scratch_types=[
            pltpu.VMEM((_P1, 128), jnp.int32),
            pltpu.VMEM((_P1, 128), jnp.int32),
            pltpu.VMEM((_P2, 128), jnp.int32),
            pltpu.VMEM((_P2, 128), jnp.int32),
            pltpu.VMEM((_P1, 128, _D), jnp.float32),
            pltpu.VMEM((_P2, 128, _O), jnp.float32),
            pltpu.SemaphoreType.DMA,
        ],
        compiler_params=pltpu.CompilerParams(use_tc_tiling_on_sc=False,
                                             needs_layout_passes=False),
    )
    def gather_kernel(t1_ref, p2_ref, x2_ref, y2_ref, o1_ref, o2_ref, xi_ref,
                      nat_v, idx_v, nat2_v, idx2_v, rows1_v, rows2_v, sem):
        wid = lax.axis_index("s") * 2 + lax.axis_index("c")

        def body1(i, carry):
            c = wid * _CH1_W + i
            line0 = c * 128
            pltpu.sync_copy(x2_ref.at[pl.ds(_P1 * c, _P1)], nat_v)
            _deinterleave(nat_v, idx_v, _P1, _P1)
            descs = [
                pltpu.async_copy(t1_ref.at[idx_v.at[j]], rows1_v.at[j], sem)
                for j in range(_P1)
            ]
            for d in descs:
                d.wait()
            for j in range(_P1):
                pltpu.sync_copy(
                    rows1_v.at[j],
                    o1_ref.at[pl.ds(line0, 128), pl.ds(_D * j, _D)])
            pltpu.sync_copy(idx_v, xi_ref.at[:, pl.ds(line0, 128)])
            return carry

        def body2(i, carry):
            c = wid * _CH2_W + i
            line0 = c * 128
            pltpu.sync_copy(y2_ref.at[pl.ds(_P2 * c, _P2)], nat2_v)
            _deinterleave(nat2_v, idx2_v, _P2, _P2)
            descs = [
                pltpu.async_copy(p2_ref.at[idx2_v.at[j]], rows2_v.at[j], sem)
                for j in range(_P2)
            ]
            for d in descs:
                d.wait()
            for j in range(_P2):
                pltpu.sync_copy(
                    rows2_v.at[j],
                    o2_ref.at[pl.ds(line0, 128), pl.ds(_O * j, _O)])
            return carry

        lax.fori_loop(0, _CH1_W, body1, 0)
        lax.fori_loop(0, _CH2_W, body2, 0)

    return gather_kernel(emb1, p2, x2, y2)


def _tc_precompute_p2(emb2, w_t):
    blk = 10000
    grid = _V2 // blk

    def p2_kernel(e_ref, w_ref, o_ref):
        o = jnp.maximum(
            jnp.dot(e_ref[...], w_ref[...],
                    preferred_element_type=jnp.float32), 0.0)
        rid = lax.broadcasted_iota(jnp.int32, (blk, _O), 0)
        first = pl.program_id(0) == 0
        o_ref[...] = jnp.where(jnp.logical_and(first, rid == 0), 0.0, o)

    return pl.pallas_call(
        p2_kernel,
        grid=(grid,),
        in_specs=[
            pl.BlockSpec((blk, _D), lambda i: (i, 0)),
            pl.BlockSpec((_D, _O), lambda i: (0, 0)),
        ],
        out_specs=pl.BlockSpec((blk, _O), lambda i: (i, 0)),
        out_shape=jax.ShapeDtypeStruct((_V2, _O), jnp.float32),
    )(emb2, w_t)


def _tc_linear_relu_packed(gw, xi, w_t):
    lines_per_blk = 1024
    grid = _LINES1 // lines_per_blk

    bd = jnp.zeros((_P1, 128, _O), jnp.float32)
    for j in range(_P1):
        bd = bd.at[j, _D * j:_D * (j + 1), :].set(w_t)

    def mm_kernel(g_ref, i_ref, w_ref, o_ref):
        gwb = g_ref[...]
        mt = jnp.transpose(i_ref[...], (1, 0)) != 0
        for j in range(_P1):
            o = jnp.maximum(
                jnp.dot(gwb, w_ref[j], preferred_element_type=jnp.float32),
                0.0)
            o_ref[:, pl.ds(_O * j, _O)] = jnp.where(mt[:, j:j + 1], o, 0.0)

    return pl.pallas_call(
        mm_kernel,
        grid=(grid,),
        in_specs=[
            pl.BlockSpec((lines_per_blk, 128), lambda i: (i, 0)),
            pl.BlockSpec((_P1, lines_per_blk), lambda i: (0, i)),
            pl.BlockSpec((_P1, 128, _O), lambda i: (0, 0, 0)),
        ],
        out_specs=pl.BlockSpec((lines_per_blk, _P1 * _O), lambda i: (i, 0)),
        out_shape=jax.ShapeDtypeStruct((_LINES1, _P1 * _O), jnp.float32),
    )(gw, xi, bd)


def kernel(x, y, emb1, emb2, W1, W2):
    x2 = x.reshape(_N // 128, 128)
    y2 = y.reshape(_N // 128, 128)

    p2_table = _tc_precompute_p2(emb2, W2.T)
    g1, q2, xi = _sc_gather(emb1, p2_table, x2, y2)

    p1 = _tc_linear_relu_packed(g1, xi, W1.T)

    o1 = p1.reshape(_B, _L, _O)
    o2 = q2.reshape(_B, _L, _O)
    return (o1, o2)


# speedup vs baseline: 2.0496x; 1.0289x over previous
import functools

import jax
import jax.numpy as jnp
from jax import lax
from jax.experimental import pallas as pl
from jax.experimental.pallas import tpu as pltpu
from jax.experimental.pallas import tpu_sc as plsc

_B = 16384
_L = 50
_N = _B * _L
_V2 = 100000
_D = 32
_O = 64
_NW = 32

_P1 = 4
_LINES1 = _N // _P1
_CH1 = _LINES1 // 128
_CH1_W = _CH1 // _NW

_P2 = 2
_LINES2 = _N // _P2
_CH2 = _LINES2 // 128
_CH2_W = _CH2 // _NW


def _deinterleave(src_ref, dst_ref, stride, nstreams):
    for j in range(nstreams):
        for g in range(8):
            il = lax.iota(jnp.int32, 16) + 16 * g
            f = stride * il + j
            vals = plsc.load_gather(
                src_ref, [lax.shift_right_logical(f, 7),
                          lax.bitwise_and(f, 127)])
            dst_ref[j, pl.ds(16 * g, 16)] = vals


def _sc_gather(emb1, p2, x2, y2):
    mesh = plsc.VectorSubcoreMesh(core_axis_name="c", subcore_axis_name="s")

    @functools.partial(
        pl.kernel,
        out_type=(
            jax.ShapeDtypeStruct((_LINES1, 128), jnp.float32),
            jax.ShapeDtypeStruct((_LINES2, 128), jnp.float32),
            jax.ShapeDtypeStruct((_P1, _LINES1), jnp.int32),
        ),
        mesh=mesh,
        scratch_types=[
            pltpu.VMEM((2, _P1, 128), jnp.int32),
            pltpu.VMEM((2, _P1, 128), jnp.int32),
            pltpu.VMEM((2, _P2, 128), jnp.int32),
            pltpu.VMEM((2, _P2, 128), jnp.int32),
            pltpu.VMEM((2, _P1, 128, _D), jnp.float32),
            pltpu.VMEM((2, _P2, 128, _O), jnp.float32),
            pltpu.SemaphoreType.DMA,
        ],
        compiler_params=pltpu.CompilerParams(use_tc_tiling_on_sc=False,
                                             needs_layout_passes=False),
    )
    def gather_kernel(t1_ref, p2_ref, x2_ref, y2_ref, o1_ref, o2_ref, xi_ref,
                      nat_v, idx_v, nat2_v, idx2_v, rows1_v, rows2_v, sem):
        wid = lax.axis_index("s") * 2 + lax.axis_index("c")

        def body1(i, carry):
            for b in range(2):
                c = wid * _CH1_W + 2 * i + b
                pltpu.sync_copy(x2_ref.at[pl.ds(_P1 * c, _P1)], nat_v.at[b])
                _deinterleave(nat_v.at[b], idx_v.at[b], _P1, _P1)
            descs = [
                pltpu.async_copy(t1_ref.at[idx_v.at[b, j]],
                                 rows1_v.at[b, j], sem)
                for b in range(2) for j in range(_P1)
            ]
            for d in descs:
                d.wait()
            for b in range(2):
                line0 = (wid * _CH1_W + 2 * i + b) * 128
                for j in range(_P1):
                    pltpu.sync_copy(
                        rows1_v.at[b, j],
                        o1_ref.at[pl.ds(line0, 128), pl.ds(_D * j, _D)])
                pltpu.sync_copy(idx_v.at[b], xi_ref.at[:, pl.ds(line0, 128)])
            return carry

        def body2(i, carry):
            for b in range(2):
                c = wid * _CH2_W + 2 * i + b
                pltpu.sync_copy(y2_ref.at[pl.ds(_P2 * c, _P2)], nat2_v.at[b])
                _deinterleave(nat2_v.at[b], idx2_v.at[b], _P2, _P2)
            descs = [
                pltpu.async_copy(p2_ref.at[idx2_v.at[b, j]],
                                 rows2_v.at[b, j], sem)
                for b in range(2) for j in range(_P2)
            ]
            for d in descs:
                d.wait()
            for b in range(2):
                line0 = (wid * _CH2_W + 2 * i + b) * 128
                for j in range(_P2):
                    pltpu.sync_copy(
                        rows2_v.at[b, j],
                        o2_ref.at[pl.ds(line0, 128), pl.ds(_O * j, _O)])
            return carry

        lax.fori_loop(0, _CH1_W // 2, body1, 0)
        lax.fori_loop(0, _CH2_W // 2, body2, 0)

    return gather_kernel(emb1, p2, x2, y2)


def _tc_precompute_p2(emb2, w_t):
    blk = 10000
    grid = _V2 // blk

    def p2_kernel(e_ref, w_ref, o_ref):
        o = jnp.maximum(
            jnp.dot(e_ref[...], w_ref[...],
                    preferred_element_type=jnp.float32), 0.0)
        rid = lax.broadcasted_iota(jnp.int32, (blk, _O), 0)
        first = pl.program_id(0) == 0
        o_ref[...] = jnp.where(jnp.logical_and(first, rid == 0), 0.0, o)

    return pl.pallas_call(
        p2_kernel,
        grid=(grid,),
        in_specs=[
            pl.BlockSpec((blk, _D), lambda i: (i, 0)),
            pl.BlockSpec((_D, _O), lambda i: (0, 0)),
        ],
        out_specs=pl.BlockSpec((blk, _O), lambda i: (i, 0)),
        out_shape=jax.ShapeDtypeStruct((_V2, _O), jnp.float32),
    )(emb2, w_t)


def _tc_linear_relu_packed(gw, xi, w_t):
    lines_per_blk = 1024
    grid = _LINES1 // lines_per_blk

    bd = jnp.zeros((_P1, 128, _O), jnp.float32)
    for j in range(_P1):
        bd = bd.at[j, _D * j:_D * (j + 1), :].set(w_t)

    def mm_kernel(g_ref, i_ref, w_ref, o_ref):
        gwb = g_ref[...]
        mt = jnp.transpose(i_ref[...], (1, 0)) != 0
        for j in range(_P1):
            o = jnp.maximum(
                jnp.dot(gwb, w_ref[j], preferred_element_type=jnp.float32),
                0.0)
            o_ref[:, pl.ds(_O * j, _O)] = jnp.where(mt[:, j:j + 1], o, 0.0)

    return pl.pallas_call(
        mm_kernel,
        grid=(grid,),
        in_specs=[
            pl.BlockSpec((lines_per_blk, 128), lambda i: (i, 0)),
            pl.BlockSpec((_P1, lines_per_blk), lambda i: (0, i)),
            pl.BlockSpec((_P1, 128, _O), lambda i: (0, 0, 0)),
        ],
        out_specs=pl.BlockSpec((lines_per_blk, _P1 * _O), lambda i: (i, 0)),
        out_shape=jax.ShapeDtypeStruct((_LINES1, _P1 * _O), jnp.float32),
    )(gw, xi, bd)


def kernel(x, y, emb1, emb2, W1, W2):
    x2 = x.reshape(_N // 128, 128)
    y2 = y.reshape(_N // 128, 128)

    p2_table = _tc_precompute_p2(emb2, W2.T)
    g1, q2, xi = _sc_gather(emb1, p2_table, x2, y2)

    p1 = _tc_linear_relu_packed(g1, xi, W1.T)

    o1 = p1.reshape(_B, _L, _O)
    o2 = q2.reshape(_B, _L, _O)
    return (o1, o2)


# speedup vs baseline: 2.3847x; 1.1635x over previous
import functools

import jax
import jax.numpy as jnp
from jax import lax
from jax.experimental import pallas as pl
from jax.experimental.pallas import tpu as pltpu
from jax.experimental.pallas import tpu_sc as plsc

_B = 16384
_L = 50
_N = _B * _L
_V2 = 100000
_D = 32
_O = 64
_NW = 32

_P1 = 4
_LINES1 = _N // _P1
_CH1 = _LINES1 // 128
_CH1_W = _CH1 // _NW

_P2 = 2
_LINES2 = _N // _P2
_CH2 = _LINES2 // 128
_CH2_W = _CH2 // _NW


def _deinterleave(src_ref, dst_ref, stride, nstreams):
    for j in range(nstreams):
        for g in range(8):
            il = lax.iota(jnp.int32, 16) + 16 * g
            f = stride * il + j
            vals = plsc.load_gather(
                src_ref, [lax.shift_right_logical(f, 7),
                          lax.bitwise_and(f, 127)])
            dst_ref[j, pl.ds(16 * g, 16)] = vals


def _sc_gather1(emb1, x2):
    mesh = plsc.VectorSubcoreMesh(core_axis_name="c", subcore_axis_name="s")

    @functools.partial(
        pl.kernel,
        out_type=(
            jax.ShapeDtypeStruct((_LINES1, 128), jnp.float32),
            jax.ShapeDtypeStruct((_P1, _LINES1), jnp.int32),
        ),
        mesh=mesh,
        scratch_types=[
            pltpu.VMEM((2, _P1, 128), jnp.int32),
            pltpu.VMEM((2, _P1, 128), jnp.int32),
            pltpu.VMEM((2, _P1, 128, _D), jnp.float32),
            pltpu.SemaphoreType.DMA,
        ],
        compiler_params=pltpu.CompilerParams(use_tc_tiling_on_sc=False,
                                             needs_layout_passes=False),
    )
    def gather_kernel(t1_ref, x2_ref, o1_ref, xi_ref,
                      nat_v, idx_v, rows1_v, sem):
        wid = lax.axis_index("s") * 2 + lax.axis_index("c")

        def body1(i, carry):
            for b in range(2):
                c = wid * _CH1_W + 2 * i + b
                pltpu.sync_copy(x2_ref.at[pl.ds(_P1 * c, _P1)], nat_v.at[b])
                _deinterleave(nat_v.at[b], idx_v.at[b], _P1, _P1)
            descs = [
                pltpu.async_copy(t1_ref.at[idx_v.at[b, j]],
                                 rows1_v.at[b, j], sem)
                for b in range(2) for j in range(_P1)
            ]
            for d in descs:
                d.wait()
            for b in range(2):
                line0 = (wid * _CH1_W + 2 * i + b) * 128
                for j in range(_P1):
                    pltpu.sync_copy(
                        rows1_v.at[b, j],
                        o1_ref.at[pl.ds(line0, 128), pl.ds(_D * j, _D)])
                pltpu.sync_copy(idx_v.at[b], xi_ref.at[:, pl.ds(line0, 128)])
            return carry

        lax.fori_loop(0, _CH1_W // 2, body1, 0)

    return gather_kernel(emb1, x2)


def _sc_gather2(p2, y2):
    mesh = plsc.VectorSubcoreMesh(core_axis_name="c", subcore_axis_name="s")

    @functools.partial(
        pl.kernel,
        out_type=jax.ShapeDtypeStruct((_LINES2, 128), jnp.float32),
        mesh=mesh,
        scratch_types=[
            pltpu.VMEM((2, _P2, 128), jnp.int32),
            pltpu.VMEM((2, _P2, 128), jnp.int32),
            pltpu.VMEM((2, _P2, 128, _O), jnp.float32),
            pltpu.SemaphoreType.DMA,
        ],
        compiler_params=pltpu.CompilerParams(use_tc_tiling_on_sc=False,
                                             needs_layout_passes=False),
    )
    def gather_kernel(p2_ref, y2_ref, o2_ref, nat2_v, idx2_v, rows2_v, sem):
        wid = lax.axis_index("s") * 2 + lax.axis_index("c")

        def body2(i, carry):
            for b in range(2):
                c = wid * _CH2_W + 2 * i + b
                pltpu.sync_copy(y2_ref.at[pl.ds(_P2 * c, _P2)], nat2_v.at[b])
                _deinterleave(nat2_v.at[b], idx2_v.at[b], _P2, _P2)
            descs = [
                pltpu.async_copy(p2_ref.at[idx2_v.at[b, j]],
                                 rows2_v.at[b, j], sem)
                for b in range(2) for j in range(_P2)
            ]
            for d in descs:
                d.wait()
            for b in range(2):
                line0 = (wid * _CH2_W + 2 * i + b) * 128
                for j in range(_P2):
                    pltpu.sync_copy(
                        rows2_v.at[b, j],
                        o2_ref.at[pl.ds(line0, 128), pl.ds(_O * j, _O)])
            return carry

        lax.fori_loop(0, _CH2_W // 2, body2, 0)

    return gather_kernel(p2, y2)


def _tc_precompute_p2(emb2, w_t):
    blk = 10000
    grid = _V2 // blk

    def p2_kernel(e_ref, w_ref, o_ref):
        o = jnp.maximum(
            jnp.dot(e_ref[...], w_ref[...],
                    preferred_element_type=jnp.float32), 0.0)
        rid = lax.broadcasted_iota(jnp.int32, (blk, _O), 0)
        first = pl.program_id(0) == 0
        o_ref[...] = jnp.where(jnp.logical_and(first, rid == 0), 0.0, o)

    return pl.pallas_call(
        p2_kernel,
        grid=(grid,),
        in_specs=[
            pl.BlockSpec((blk, _D), lambda i: (i, 0)),
            pl.BlockSpec((_D, _O), lambda i: (0, 0)),
        ],
        out_specs=pl.BlockSpec((blk, _O), lambda i: (i, 0)),
        out_shape=jax.ShapeDtypeStruct((_V2, _O), jnp.float32),
    )(emb2, w_t)


def _tc_linear_relu_packed(gw, xi, w_t):
    lines_per_blk = 1024
    grid = _LINES1 // lines_per_blk

    bd = jnp.zeros((_P1, 128, _O), jnp.float32)
    for j in range(_P1):
        bd = bd.at[j, _D * j:_D * (j + 1), :].set(w_t)

    def mm_kernel(g_ref, i_ref, w_ref, o_ref):
        gwb = g_ref[...]
        mt = jnp.transpose(i_ref[...], (1, 0)) != 0
        for j in range(_P1):
            o = jnp.maximum(
                jnp.dot(gwb, w_ref[j], preferred_element_type=jnp.float32),
                0.0)
            o_ref[:, pl.ds(_O * j, _O)] = jnp.where(mt[:, j:j + 1], o, 0.0)

    return pl.pallas_call(
        mm_kernel,
        grid=(grid,),
        in_specs=[
            pl.BlockSpec((lines_per_blk, 128), lambda i: (i, 0)),
            pl.BlockSpec((_P1, lines_per_blk), lambda i: (0, i)),
            pl.BlockSpec((_P1, 128, _O), lambda i: (0, 0, 0)),
        ],
        out_specs=pl.BlockSpec((lines_per_blk, _P1 * _O), lambda i: (i, 0)),
        out_shape=jax.ShapeDtypeStruct((_LINES1, _P1 * _O), jnp.float32),
    )(gw, xi, bd)


def kernel(x, y, emb1, emb2, W1, W2):
    x2 = x.reshape(_N // 128, 128)
    y2 = y.reshape(_N // 128, 128)

    g1, xi = _sc_gather1(emb1, x2)
    p2_table = _tc_precompute_p2(emb2, W2.T)
    q2 = _sc_gather2(p2_table, y2)

    p1 = _tc_linear_relu_packed(g1, xi, W1.T)

    o1 = p1.reshape(_B, _L, _O)
    o2 = q2.reshape(_B, _L, _O)
    return (o1, o2)
